# baseline (speedup 1.0000x reference)
import functools

import jax
import jax.numpy as jnp
from jax import lax
from jax.experimental import pallas as pl
from jax.experimental.pallas import tpu as pltpu
from jax.experimental.pallas import tpu_sc as plsc

_EMBED = 100
_DPAD = 112
_PROJ = 200
_NC = 2
_NS = 16
_NW = _NC * _NS
_CHUNK = 128
_BM = 2048


def _gather_rows(table, idx_flat):
    n = idx_flat.shape[0]
    d = table.shape[1]
    n_per_w = n // _NW
    steps = n_per_w // _CHUNK
    mesh = plsc.VectorSubcoreMesh(core_axis_name="c", subcore_axis_name="s")

    @functools.partial(
        pl.kernel,
        out_type=jax.ShapeDtypeStruct((n, d), table.dtype),
        mesh=mesh,
        compiler_params=pltpu.CompilerParams(use_tc_tiling_on_sc=False),
        scratch_types=[
            pltpu.VMEM((_CHUNK,), jnp.int32),
            pltpu.VMEM((_CHUNK, d), jnp.float32),
            pltpu.SemaphoreType.DMA,
        ],
    )
    def k(table_hbm, idx_hbm, out_hbm, idx_v, rows_v, sem):
        wid = lax.axis_index("s") * _NC + lax.axis_index("c")
        base = wid * n_per_w

        def body(j, carry):
            off = base + j * _CHUNK
            pltpu.sync_copy(idx_hbm.at[pl.ds(off, _CHUNK)], idx_v)
            pltpu.async_copy(table_hbm.at[idx_v], rows_v, sem).wait()
            pltpu.sync_copy(rows_v, out_hbm.at[pl.ds(off, _CHUNK)])
            return carry

        lax.fori_loop(0, steps, body, 0)

    return k(table, idx_flat)


def _project(rows, W, b):
    m, d = rows.shape

    def body(x_ref, w_ref, b_ref, o_ref):
        o_ref[...] = (
            jnp.dot(x_ref[...], w_ref[...], preferred_element_type=jnp.float32)
            + b_ref[...]
        )

    return pl.pallas_call(
        body,
        grid=(m // _BM,),
        in_specs=[
            pl.BlockSpec((_BM, d), lambda i: (i, 0)),
            pl.BlockSpec((d, _PROJ), lambda i: (0, 0)),
            pl.BlockSpec((1, _PROJ), lambda i: (0, 0)),
        ],
        out_specs=pl.BlockSpec((_BM, _PROJ), lambda i: (i, 0)),
        out_shape=jax.ShapeDtypeStruct((m, _PROJ), jnp.float32),
    )(rows, W, b.reshape(1, _PROJ))


def kernel(premise_w_data, premise_w_lens, hyp_w_data, hyp_w_lens, tag_b, table, W, b):
    B, L = premise_w_data.shape
    p_idx = premise_w_data.reshape(-1).astype(jnp.int32)
    h_idx = hyp_w_data.reshape(-1).astype(jnp.int32)
    table_p = jnp.pad(table, ((0, 0), (0, _DPAD - _EMBED)))
    W_p = jnp.pad(W, ((0, _DPAD - _EMBED), (0, 0)))
    p_rows = _gather_rows(table_p, p_idx)
    h_rows = _gather_rows(table_p, h_idx)
    p_out = _project(p_rows, W_p, b).reshape(B, L, _PROJ)
    h_out = _project(h_rows, W_p, b).reshape(B, L, _PROJ)
    return (p_out, h_out)


# speedup vs baseline: 1.2258x; 1.2258x over previous
import functools

import jax
import jax.numpy as jnp
from jax import lax
from jax.experimental import pallas as pl
from jax.experimental.pallas import tpu as pltpu
from jax.experimental.pallas import tpu_sc as plsc

_EMBED = 100
_DPAD = 128
_PROJ = 200
_NC = 2
_NS = 16
_NW = _NC * _NS
_CHUNK = 128
_BM = 2048


def _gather_rows(table, idx_flat):
    n = idx_flat.shape[0]
    d = table.shape[1]
    n_per_w = n // _NW
    steps = n_per_w // _CHUNK
    mesh = plsc.VectorSubcoreMesh(core_axis_name="c", subcore_axis_name="s")

    @functools.partial(
        pl.kernel,
        out_type=jax.ShapeDtypeStruct((n, d), table.dtype),
        mesh=mesh,
        scratch_types=[
            pltpu.VMEM((n_per_w,), jnp.int32),
            pltpu.VMEM((_CHUNK, d), jnp.float32),
            pltpu.SemaphoreType.DMA,
        ],
    )
    def k(table_hbm, idx_hbm, out_hbm, idx_v, rows_v, sem):
        wid = lax.axis_index("s") * _NC + lax.axis_index("c")
        base = wid * n_per_w
        pltpu.sync_copy(idx_hbm.at[pl.ds(base, n_per_w)], idx_v)

        def body(j, carry):
            pltpu.async_copy(
                table_hbm.at[idx_v.at[pl.ds(j * _CHUNK, _CHUNK)]], rows_v, sem
            ).wait()
            pltpu.sync_copy(rows_v, out_hbm.at[pl.ds(base + j * _CHUNK, _CHUNK)])
            return carry

        lax.fori_loop(0, steps, body, 0)

    return k(table, idx_flat)


def _project(rows, W, b):
    m, d = rows.shape

    def body(x_ref, w_ref, b_ref, o_ref):
        o_ref[...] = (
            jnp.dot(x_ref[...], w_ref[...], preferred_element_type=jnp.float32)
            + b_ref[...]
        )

    return pl.pallas_call(
        body,
        grid=(m // _BM,),
        in_specs=[
            pl.BlockSpec((_BM, d), lambda i: (i, 0)),
            pl.BlockSpec((d, _PROJ), lambda i: (0, 0)),
            pl.BlockSpec((1, _PROJ), lambda i: (0, 0)),
        ],
        out_specs=pl.BlockSpec((_BM, _PROJ), lambda i: (i, 0)),
        out_shape=jax.ShapeDtypeStruct((m, _PROJ), jnp.float32),
    )(rows, W, b.reshape(1, _PROJ))


def kernel(premise_w_data, premise_w_lens, hyp_w_data, hyp_w_lens, tag_b, table, W, b):
    B, L = premise_w_data.shape
    p_idx = premise_w_data.reshape(-1).astype(jnp.int32)
    h_idx = hyp_w_data.reshape(-1).astype(jnp.int32)
    table_p = jnp.pad(table, ((0, 0), (0, _DPAD - _EMBED)))
    W_p = jnp.pad(W, ((0, _DPAD - _EMBED), (0, 0)))
    p_rows = _gather_rows(table_p, p_idx)
    h_rows = _gather_rows(table_p, h_idx)
    p_out = _project(p_rows, W_p, b).reshape(B, L, _PROJ)
    h_out = _project(h_rows, W_p, b).reshape(B, L, _PROJ)
    return (p_out, h_out)


# speedup vs baseline: 2.2517x; 1.8368x over previous
import functools

import jax
import jax.numpy as jnp
from jax import lax
from jax.experimental import pallas as pl
from jax.experimental.pallas import tpu as pltpu
from jax.experimental.pallas import tpu_sc as plsc

_EMBED = 100
_DPAD = 128
_PROJ = 200
_B = 4096
_L = 50
_NC = 2
_NS = 16
_NW = _NC * _NS
_CHUNK = 128
_BC = 512
_VPAD = 400384
_BN = 1024


def _format(tableT):
    grid = _VPAD // _BC

    def body(x_ref, o_ref):
        xt = x_ref[...].T
        pad = jnp.zeros((_BC, _DPAD - _EMBED - 1), jnp.float32)
        one = jnp.ones((_BC, 1), jnp.float32)
        o_ref[...] = jnp.concatenate([xt, pad, one], axis=1)

    return pl.pallas_call(
        body,
        grid=(grid,),
        in_specs=[pl.BlockSpec((_EMBED, _BC), lambda i: (0, i))],
        out_specs=pl.BlockSpec((_BC, _DPAD), lambda i: (i, 0)),
        out_shape=jax.ShapeDtypeStruct((_VPAD, _DPAD), jnp.float32),
    )(tableT)


def _gather_rows(table, idx_flat):
    n = idx_flat.shape[0]
    d = table.shape[1]
    n_per_w = n // _NW
    steps = n_per_w // _CHUNK
    mesh = plsc.VectorSubcoreMesh(core_axis_name="c", subcore_axis_name="s")

    @functools.partial(
        pl.kernel,
        out_type=jax.ShapeDtypeStruct((n, d), table.dtype),
        mesh=mesh,
        scratch_types=[
            pltpu.VMEM((n_per_w,), jnp.int32),
            pltpu.VMEM((_CHUNK, d), jnp.float32),
            pltpu.SemaphoreType.DMA,
        ],
    )
    def k(table_hbm, idx_hbm, out_hbm, idx_v, rows_v, sem):
        wid = lax.axis_index("s") * _NC + lax.axis_index("c")
        base = wid * n_per_w
        pltpu.sync_copy(idx_hbm.at[pl.ds(base, n_per_w)], idx_v)

        def body(j, carry):
            pltpu.async_copy(
                table_hbm.at[idx_v.at[pl.ds(j * _CHUNK, _CHUNK)]], rows_v, sem
            ).wait()
            pltpu.sync_copy(rows_v, out_hbm.at[pl.ds(base + j * _CHUNK, _CHUNK)])
            return carry

        lax.fori_loop(0, steps, body, 0)

    return k(table, idx_flat)


def _project(rows, W_aug):
    def body(x_ref, w_ref, o_ref):
        acc = lax.dot_general(
            w_ref[...], x_ref[...],
            (((0,), (1,)), ((), ())),
            preferred_element_type=jnp.float32,
        )
        o_ref[...] = acc[None]

    return pl.pallas_call(
        body,
        grid=(_L, _B // _BN),
        in_specs=[
            pl.BlockSpec((_BN, _DPAD), lambda l, n: (l * (_B // _BN) + n, 0)),
            pl.BlockSpec((_DPAD, _PROJ), lambda l, n: (0, 0)),
        ],
        out_specs=pl.BlockSpec((1, _PROJ, _BN), lambda l, n: (l, 0, n)),
        out_shape=jax.ShapeDtypeStruct((_L, _PROJ, _B), jnp.float32),
    )(rows, W_aug)


def kernel(premise_w_data, premise_w_lens, hyp_w_data, hyp_w_lens, tag_b, table, W, b):
    tableRM = _format(table.T)
    W_aug = jnp.concatenate(
        [W, jnp.zeros((_DPAD - _EMBED - 1, _PROJ), jnp.float32), b[None, :]], axis=0
    )
    p_idx = premise_w_data.T.reshape(-1).astype(jnp.int32)
    h_idx = hyp_w_data.T.reshape(-1).astype(jnp.int32)
    p_rows = _gather_rows(tableRM, p_idx)
    h_rows = _gather_rows(tableRM, h_idx)
    p_out = _project(p_rows, W_aug).transpose(2, 0, 1)
    h_out = _project(h_rows, W_aug).transpose(2, 0, 1)
    return (p_out, h_out)


# speedup vs baseline: 4.2041x; 1.8671x over previous
import functools

import jax
import jax.numpy as jnp
from jax import lax
from jax.experimental import pallas as pl
from jax.experimental.pallas import tpu as pltpu
from jax.experimental.pallas import tpu_sc as plsc

_EMBED = 100
_DPAD = 128
_PROJ = 200
_B = 4096
_L = 50
_NC = 2
_NS = 16
_NW = _NC * _NS
_CHUNK = 128
_BC = 4096
_VPAD = 401408
_BN = 2048


def _format(tableT):
    grid = _VPAD // _BC

    def body(x_ref, o_ref):
        xt = x_ref[...].T
        pad = jnp.zeros((_BC, _DPAD - _EMBED - 1), jnp.float32)
        one = jnp.ones((_BC, 1), jnp.float32)
        o_ref[...] = jnp.concatenate([xt, pad, one], axis=1)

    return pl.pallas_call(
        body,
        grid=(grid,),
        in_specs=[pl.BlockSpec((_EMBED, _BC), lambda i: (0, i))],
        out_specs=pl.BlockSpec((_BC, _DPAD), lambda i: (i, 0)),
        out_shape=jax.ShapeDtypeStruct((_VPAD, _DPAD), jnp.float32),
    )(tableT)


def _gather_rows(table, idx_flat):
    n = idx_flat.shape[0]
    d = table.shape[1]
    n_per_w = n // _NW
    steps = n_per_w // _CHUNK
    npairs = steps // 2
    mesh = plsc.VectorSubcoreMesh(core_axis_name="c", subcore_axis_name="s")

    @functools.partial(
        pl.kernel,
        out_type=jax.ShapeDtypeStruct((n, d), table.dtype),
        mesh=mesh,
        scratch_types=[
            pltpu.VMEM((n_per_w,), jnp.int32),
            pltpu.VMEM((_CHUNK, d), jnp.float32),
            pltpu.VMEM((_CHUNK, d), jnp.float32),
            pltpu.SemaphoreType.DMA,
            pltpu.SemaphoreType.DMA,
        ],
    )
    def k(table_hbm, idx_hbm, out_hbm, idx_v, rows0, rows1, sem0, sem1):
        wid = lax.axis_index("s") * _NC + lax.axis_index("c")
        base = wid * n_per_w
        pltpu.sync_copy(idx_hbm.at[pl.ds(base, n_per_w)], idx_v)

        def gth(j, buf, sem):
            return pltpu.make_async_copy(
                table_hbm.at[idx_v.at[pl.ds(j * _CHUNK, _CHUNK)]], buf, sem
            )

        gth(0, rows0, sem0).start()

        def body(jj, carry):
            j0 = jj * 2
            j1 = j0 + 1
            gth(j1, rows1, sem1).start()
            gth(j0, rows0, sem0).wait()
            pltpu.sync_copy(rows0, out_hbm.at[pl.ds(base + j0 * _CHUNK, _CHUNK)])

            @pl.when(jj + 1 < npairs)
            def _():
                gth(j0 + 2, rows0, sem0).start()

            gth(j1, rows1, sem1).wait()
            pltpu.sync_copy(rows1, out_hbm.at[pl.ds(base + j1 * _CHUNK, _CHUNK)])
            return carry

        lax.fori_loop(0, npairs, body, 0)

    return k(table, idx_flat)


def _project(rows, W_aug):
    def body(x_ref, w_ref, o_ref):
        acc = lax.dot_general(
            w_ref[...], x_ref[...],
            (((0,), (1,)), ((), ())),
            preferred_element_type=jnp.float32,
        )
        o_ref[...] = acc[None]

    return pl.pallas_call(
        body,
        grid=(_L, _B // _BN),
        in_specs=[
            pl.BlockSpec((_BN, _DPAD), lambda l, n: (l * (_B // _BN) + n, 0)),
            pl.BlockSpec((_DPAD, _PROJ), lambda l, n: (0, 0)),
        ],
        out_specs=pl.BlockSpec((1, _PROJ, _BN), lambda l, n: (l, 0, n)),
        out_shape=jax.ShapeDtypeStruct((_L, _PROJ, _B), jnp.float32),
    )(rows, W_aug)


def kernel(premise_w_data, premise_w_lens, hyp_w_data, hyp_w_lens, tag_b, table, W, b):
    tableRM = _format(table.T)
    W_aug = jnp.concatenate(
        [W, jnp.zeros((_DPAD - _EMBED - 1, _PROJ), jnp.float32), b[None, :]], axis=0
    )
    p_idx = premise_w_data.T.reshape(-1).astype(jnp.int32)
    h_idx = hyp_w_data.T.reshape(-1).astype(jnp.int32)
    p_rows = _gather_rows(tableRM, p_idx)
    h_rows = _gather_rows(tableRM, h_idx)
    p_out = _project(p_rows, W_aug).transpose(2, 0, 1)
    h_out = _project(h_rows, W_aug).transpose(2, 0, 1)
    return (p_out, h_out)


# speedup vs baseline: 4.3178x; 1.0270x over previous
import functools

import jax
import jax.numpy as jnp
from jax import lax
from jax.experimental import pallas as pl
from jax.experimental.pallas import tpu as pltpu
from jax.experimental.pallas import tpu_sc as plsc

_EMBED = 100
_DPAD = 128
_PROJ = 200
_B = 4096
_L = 50
_NC = 2
_NS = 16
_NW = _NC * _NS
_CHUNK = 128
_BC = 4096
_VPAD = 401408
_BN = 2048


def _format(tableT):
    grid = _VPAD // _BC

    def body(x_ref, o_ref):
        xt = x_ref[...].T
        pad = jnp.zeros((_BC, _DPAD - _EMBED - 1), jnp.float32)
        one = jnp.ones((_BC, 1), jnp.float32)
        o_ref[...] = jnp.concatenate([xt, pad, one], axis=1)

    return pl.pallas_call(
        body,
        grid=(grid,),
        in_specs=[pl.BlockSpec((_EMBED, _BC), lambda i: (0, i))],
        out_specs=pl.BlockSpec((_BC, _DPAD), lambda i: (i, 0)),
        out_shape=jax.ShapeDtypeStruct((_VPAD, _DPAD), jnp.float32),
    )(tableT)


def _gather_rows(table, idx_flat):
    n = idx_flat.shape[0]
    d = table.shape[1]
    n_per_w = n // _NW
    steps = n_per_w // _CHUNK
    npairs = steps // 2
    mesh = plsc.VectorSubcoreMesh(core_axis_name="c", subcore_axis_name="s")

    @functools.partial(
        pl.kernel,
        out_type=jax.ShapeDtypeStruct((n, d), table.dtype),
        mesh=mesh,
        scratch_types=[
            pltpu.VMEM((n_per_w,), jnp.int32),
            pltpu.VMEM((_CHUNK, d), jnp.float32),
            pltpu.VMEM((_CHUNK, d), jnp.float32),
            pltpu.SemaphoreType.DMA,
            pltpu.SemaphoreType.DMA,
        ],
    )
    def k(table_hbm, idx_hbm, out_hbm, idx_v, rows0, rows1, sem0, sem1):
        wid = lax.axis_index("s") * _NC + lax.axis_index("c")
        base = wid * n_per_w
        pltpu.sync_copy(idx_hbm.at[pl.ds(base, n_per_w)], idx_v)

        def gth(j, buf, sem):
            return pltpu.make_async_copy(
                table_hbm.at[idx_v.at[pl.ds(j * _CHUNK, _CHUNK)]], buf, sem
            )

        gth(0, rows0, sem0).start()

        def body(jj, carry):
            j0 = jj * 2
            j1 = j0 + 1
            gth(j1, rows1, sem1).start()
            gth(j0, rows0, sem0).wait()
            pltpu.sync_copy(rows0, out_hbm.at[pl.ds(base + j0 * _CHUNK, _CHUNK)])

            @pl.when(jj + 1 < npairs)
            def _():
                gth(j0 + 2, rows0, sem0).start()

            gth(j1, rows1, sem1).wait()
            pltpu.sync_copy(rows1, out_hbm.at[pl.ds(base + j1 * _CHUNK, _CHUNK)])
            return carry

        lax.fori_loop(0, npairs, body, 0)

    return k(table, idx_flat)


def _project_half(rows, W_aug, l_base, prev=None):
    nl = rows.shape[0] // _B
    nb = _B // _BN

    def body(x_ref, w_ref, *rest):
        o_ref = rest[-1]
        acc = lax.dot_general(
            w_ref[...], x_ref[...],
            (((0,), (1,)), ((), ())),
            preferred_element_type=jnp.float32,
        )
        o_ref[...] = acc[None]

    in_specs = [
        pl.BlockSpec((_BN, _DPAD), lambda l, n: (l * nb + n, 0)),
        pl.BlockSpec((_DPAD, _PROJ), lambda l, n: (0, 0)),
    ]
    args = [rows, W_aug]
    aliases = {}
    if prev is not None:
        in_specs.append(pl.BlockSpec(memory_space=pl.ANY))
        args.append(prev)
        aliases = {2: 0}

    return pl.pallas_call(
        body,
        grid=(nl, nb),
        in_specs=in_specs,
        out_specs=pl.BlockSpec(
            (1, _PROJ, _BN), lambda l, n, l_base=l_base: (l + l_base, 0, n)
        ),
        out_shape=jax.ShapeDtypeStruct((_L, _PROJ, _B), jnp.float32),
        input_output_aliases=aliases,
    )(*args)


def _chain(tableRM, W_aug, idx):
    half = idx.shape[0] // 2
    g1 = _gather_rows(tableRM, idx[:half])
    g2 = _gather_rows(tableRM, idx[half:])
    o1 = _project_half(g1, W_aug, 0)
    o2 = _project_half(g2, W_aug, _L // 2, prev=o1)
    return o2.transpose(2, 0, 1)


def kernel(premise_w_data, premise_w_lens, hyp_w_data, hyp_w_lens, tag_b, table, W, b):
    tableRM = _format(table.T)
    W_aug = jnp.concatenate(
        [W, jnp.zeros((_DPAD - _EMBED - 1, _PROJ), jnp.float32), b[None, :]], axis=0
    )
    p_idx = premise_w_data.T.reshape(-1).astype(jnp.int32)
    h_idx = hyp_w_data.T.reshape(-1).astype(jnp.int32)
    p_out = _chain(tableRM, W_aug, p_idx)
    h_out = _chain(tableRM, W_aug, h_idx)
    return (p_out, h_out)
